# Initial kernel scaffold; baseline (speedup 1.0000x reference)
import jax
import jax.numpy as jnp
from jax.experimental import pallas as pl


def kernel(context, query_source, g_ctx, b_ctx, g_qs, b_qs, W_q, W_kv, W_out, gamma):
    raise NotImplementedError("write your pallas kernel here")


# speedup vs baseline: 4.2364x; 4.2364x over previous
import functools

import jax
import jax.numpy as jnp
from jax.experimental import pallas as pl

HEADS = 8
DIM_HEAD = 64
DIM = 384
INNER = HEADS * DIM_HEAD
H_TOPK = 16
W_TOPK = 16
HW = 32
P = HW * HW
NEG = -1e30


def _dot(a, b, dims):
    return jax.lax.dot_general(a, b, (dims, ((), ())),
                               preferred_element_type=jnp.float32)


def _stage1_body(ctx_ref, qs_ref, g_ctx_ref, b_ctx_ref, g_qs_ref, b_qs_ref,
                 wq_ref, wkv_ref, qn_ref, kn_ref, v_ref, qsn_ref, mask_ref):
    ctx = ctx_ref[0]
    qs = qs_ref[0]

    def chan_ln(x, g, b):
        mean = jnp.mean(x, axis=0, keepdims=True)
        var = jnp.mean((x - mean) ** 2, axis=0, keepdims=True)
        return (x - mean) * jax.lax.rsqrt(var + 1e-5) * g + b

    ctxn = chan_ln(ctx, g_ctx_ref[...], b_ctx_ref[...])
    qsn = chan_ln(qs, g_qs_ref[...], b_qs_ref[...])
    qsn_ref[0] = qsn

    kv = _dot(wkv_ref[...], ctxn, (((1,), (0,))))
    q = _dot(wq_ref[...], qsn, (((1,), (0,))))
    k = kv[:INNER]
    v = kv[INNER:]
    v_ref[0] = v

    i0 = jax.lax.broadcasted_iota(jnp.int32, (P, HW), 0)
    i1 = jax.lax.broadcasted_iota(jnp.int32, (P, HW), 1)
    Rm = (i0 // HW == i1).astype(jnp.float32)
    Cm = (i0 % HW == i1).astype(jnp.float32)

    mask_rows = []
    qn_all = []
    kn_all = []
    for h in range(HEADS):
        qh = q[h * DIM_HEAD:(h + 1) * DIM_HEAD]
        kh = k[h * DIM_HEAD:(h + 1) * DIM_HEAD]
        qnorm = jnp.sqrt(jnp.sum(qh * qh, axis=0, keepdims=True))
        knorm = jnp.sqrt(jnp.sum(kh * kh, axis=0, keepdims=True))
        qh = qh / jnp.maximum(qnorm, 1e-12)
        kh = kh / jnp.maximum(knorm, 1e-12)
        qn_all.append(qh)
        kn_all.append(kh)

        q_probe = jnp.sum(jnp.abs(qh), axis=1, keepdims=True)
        s_pos = jnp.sum(jnp.abs(kh) * q_probe, axis=0, keepdims=True)

        def topk_mask(sel_mat):
            s_row = _dot(s_pos, sel_mat, ((1,), (0,)))
            s_col = _dot(sel_mat, s_pos, ((0,), (1,)))
            vi = jnp.broadcast_to(s_col, (HW, HW))
            vj = jnp.broadcast_to(s_row, (HW, HW))
            ii = jax.lax.broadcasted_iota(jnp.int32, (HW, HW), 0)
            jj = jax.lax.broadcasted_iota(jnp.int32, (HW, HW), 1)
            beats = (vj > vi) | ((vj == vi) & (jj < ii))
            rank = jnp.sum(beats.astype(jnp.float32), axis=1, keepdims=True)
            sel = (rank < H_TOPK).astype(jnp.float32)
            return _dot(sel, sel_mat, ((0,), (1,)))

        mh = topk_mask(Rm)
        mw = topk_mask(Cm)
        mask_rows.append(mh * mw)

    qn_ref[0] = jnp.concatenate(qn_all, axis=0)
    kn_ref[0] = jnp.concatenate(kn_all, axis=0)
    mask_ref[0] = jnp.concatenate(mask_rows, axis=0)


def _stage2_body(qn_ref, kn_ref, v_ref, mask_ref, out_ref):
    qh = qn_ref[0]
    kh = kn_ref[0]
    vh = v_ref[0]
    mask = mask_ref[0][0:1]
    sim = _dot(qh, kh, ((0,), (0,)))
    sim = sim + (mask - 1.0) * (-NEG)
    m = jnp.max(sim, axis=1, keepdims=True)
    e = jnp.exp(sim - m)
    s = jnp.sum(e, axis=1, keepdims=True)
    attn = e / s
    out_ref[0] = _dot(vh, attn, ((1,), (1,)))


def _stage3_body(inner_ref, qsn_ref, wout_ref, gamma_ref, out_ref):
    proj = _dot(wout_ref[...], inner_ref[0], ((1,), (0,)))
    out_ref[0] = gamma_ref[0, 0] * proj + qsn_ref[0]


def kernel(context, query_source, g_ctx, b_ctx, g_qs, b_qs, W_q, W_kv, W_out,
           gamma):
    b = context.shape[0]
    B = b * HEADS
    ctx = context.reshape(b, DIM, P)
    qs = query_source.reshape(b, DIM, P)
    g_ctx = g_ctx.reshape(DIM, 1)
    b_ctx = b_ctx.reshape(DIM, 1)
    g_qs = g_qs.reshape(DIM, 1)
    b_qs = b_qs.reshape(DIM, 1)

    full = lambda shape: pl.BlockSpec(shape, lambda i: (0,) * len(shape))
    batch3 = lambda shape: pl.BlockSpec(shape, lambda i: (i, 0, 0))

    qn, kn, v, qsn, mask = pl.pallas_call(
        _stage1_body,
        grid=(b,),
        in_specs=[
            batch3((1, DIM, P)), batch3((1, DIM, P)),
            full((DIM, 1)), full((DIM, 1)), full((DIM, 1)), full((DIM, 1)),
            full((INNER, DIM)), full((2 * INNER, DIM)),
        ],
        out_specs=[
            batch3((1, INNER, P)), batch3((1, INNER, P)),
            batch3((1, INNER, P)), batch3((1, DIM, P)),
            batch3((1, HEADS, P)),
        ],
        out_shape=[
            jax.ShapeDtypeStruct((b, INNER, P), jnp.float32),
            jax.ShapeDtypeStruct((b, INNER, P), jnp.float32),
            jax.ShapeDtypeStruct((b, INNER, P), jnp.float32),
            jax.ShapeDtypeStruct((b, DIM, P), jnp.float32),
            jax.ShapeDtypeStruct((b, HEADS, P), jnp.float32),
        ],
    )(ctx, qs, g_ctx, b_ctx, g_qs, b_qs, W_q, W_kv)

    qn = qn.reshape(B, DIM_HEAD, P)
    kn = kn.reshape(B, DIM_HEAD, P)
    v = v.reshape(B, DIM_HEAD, P)
    mask8 = jnp.broadcast_to(mask.reshape(B, 1, P), (B, 8, P))

    attn_out = pl.pallas_call(
        _stage2_body,
        grid=(B,),
        in_specs=[
            batch3((1, DIM_HEAD, P)), batch3((1, DIM_HEAD, P)),
            batch3((1, DIM_HEAD, P)), batch3((1, 8, P)),
        ],
        out_specs=batch3((1, DIM_HEAD, P)),
        out_shape=jax.ShapeDtypeStruct((B, DIM_HEAD, P), jnp.float32),
    )(qn, kn, v, mask8)

    inner = attn_out.reshape(b, INNER, P)
    out = pl.pallas_call(
        _stage3_body,
        grid=(b,),
        in_specs=[
            batch3((1, INNER, P)), batch3((1, DIM, P)),
            full((DIM, INNER)), full((1, 1)),
        ],
        out_specs=batch3((1, DIM, P)),
        out_shape=jax.ShapeDtypeStruct((b, DIM, P), jnp.float32),
    )(inner, qsn, W_out, gamma.reshape(1, 1))

    return out.reshape(b, DIM, HW, HW)
